# baseline (speedup 1.0000x reference)
import dataclasses
import functools

import jax
import jax.numpy as jnp
from jax import lax
from jax.experimental import pallas as pl
from jax.experimental.pallas import tpu as pltpu
from jax.experimental.pallas import tpu_sc as plsc

N = 16384
C = 200
NC = 2
NS = 16
NW = NC * NS
L = 16
ROWS_PER_W = N // NW
CHUNK = 64
NCHUNKS = ROWS_PER_W // CHUNK
FLAT = CHUNK * C
NBUF = 2
GROUP_OFFS = tuple(range(0, C - L + 1, L)) + (C - L,)

_mesh = plsc.VectorSubcoreMesh(core_axis_name="c", subcore_axis_name="s")

_cp = pltpu.CompilerParams()
if "needs_layout_passes" in pltpu.CompilerParams.__dataclass_fields__:
  _cp = dataclasses.replace(_cp, needs_layout_passes=False)

_buf_types = []
for _ in range(NBUF):
  _buf_types += [
      pltpu.VMEM((CHUNK, C), jnp.float32),
      pltpu.VMEM((CHUNK, C), jnp.int32),
      pltpu.VMEM((CHUNK, C), jnp.float32),
      pltpu.SemaphoreType.DMA,
      pltpu.SemaphoreType.DMA,
      pltpu.SemaphoreType.DMA,
  ]


@jax.jit
def _gather_sc(x, d):
  @functools.partial(
      pl.kernel,
      out_type=jax.ShapeDtypeStruct((N, C), jnp.float32),
      mesh=_mesh,
      scratch_types=_buf_types,
      compiler_params=_cp,
  )
  def k(x_hbm, d_hbm, o_hbm, *bufs_flat):
    wid = lax.axis_index("s") * NC + lax.axis_index("c")
    base = wid * ROWS_PER_W
    lane = lax.iota(jnp.int32, L)
    bufs = [bufs_flat[6 * b:6 * (b + 1)] for b in range(NBUF)]

    pend_in = {}
    pend_out = {}

    def issue_in(cc):
      xv, dv, _, sx, sd, _ = bufs[cc % NBUF]
      r0 = base + cc * CHUNK
      pend_in[cc] = (
          pltpu.async_copy(x_hbm.at[pl.ds(r0, CHUNK)], xv, sx),
          pltpu.async_copy(d_hbm.at[pl.ds(r0, CHUNK)], dv, sd),
      )

    for cc in range(NBUF):
      issue_in(cc)

    cols = [lane + o for o in GROUP_OFFS]

    for cc in range(NCHUNKS):
      xv, dv, ov, _, _, so = bufs[cc % NBUF]
      cpx, cpd = pend_in.pop(cc)
      cpx.wait()
      cpd.wait()
      if cc - NBUF >= 0:
        pend_out.pop(cc - NBUF).wait()

      @plsc.parallel_loop(0, CHUNK, unroll=4)
      def _(r):
        rsplat = jnp.zeros((L,), jnp.int32) + r
        for col in cols:
          idx = plsc.load_gather(dv, [rsplat, col])
          vals = plsc.load_gather(xv, [rsplat, idx])
          plsc.store_scatter(ov, [rsplat, col], vals)

      r0 = base + cc * CHUNK
      pend_out[cc] = pltpu.async_copy(ov, o_hbm.at[pl.ds(r0, CHUNK)], so)
      if cc + NBUF < NCHUNKS:
        issue_in(cc + NBUF)

    for cc in range(NCHUNKS - NBUF, NCHUNKS):
      pend_out.pop(cc).wait()

  return k(x, d)


def kernel(x, d):
  return _gather_sc(x, d.astype(jnp.int32))


# speedup vs baseline: 1.0163x; 1.0163x over previous
import dataclasses
import functools

import jax
import jax.numpy as jnp
from jax import lax
from jax.experimental import pallas as pl
from jax.experimental.pallas import tpu as pltpu
from jax.experimental.pallas import tpu_sc as plsc

N = 16384
C = 200
NC = 2
NS = 16
NW = NC * NS
L = 16
ROWS_PER_W = N // NW
CHUNK = 64
NCHUNKS = ROWS_PER_W // CHUNK
FLAT = CHUNK * C
NBUF = 2
GROUP_OFFS = tuple(range(0, C - L + 1, L)) + (C - L,)

_mesh = plsc.VectorSubcoreMesh(core_axis_name="c", subcore_axis_name="s")

_cp = pltpu.CompilerParams()
if "needs_layout_passes" in pltpu.CompilerParams.__dataclass_fields__:
  _cp = dataclasses.replace(_cp, needs_layout_passes=False)

_buf_types = []
for _ in range(NBUF):
  _buf_types += [
      pltpu.VMEM((CHUNK, C), jnp.float32),
      pltpu.VMEM((CHUNK, C), jnp.int32),
      pltpu.VMEM((CHUNK, C), jnp.float32),
      pltpu.SemaphoreType.DMA,
      pltpu.SemaphoreType.DMA,
      pltpu.SemaphoreType.DMA,
  ]


@jax.jit
def _gather_sc(x, d):
  @functools.partial(
      pl.kernel,
      out_type=jax.ShapeDtypeStruct((N, C), jnp.float32),
      mesh=_mesh,
      scratch_types=_buf_types,
      compiler_params=_cp,
  )
  def k(x_hbm, d_hbm, o_hbm, *bufs_flat):
    wid = lax.axis_index("s") * NC + lax.axis_index("c")
    base = wid * ROWS_PER_W
    lane = lax.iota(jnp.int32, L)
    bufs = [bufs_flat[6 * b:6 * (b + 1)] for b in range(NBUF)]

    pend_in = {}
    pend_out = {}

    def issue_in(cc):
      xv, dv, _, sx, sd, _ = bufs[cc % NBUF]
      r0 = base + cc * CHUNK
      pend_in[cc] = (
          pltpu.async_copy(x_hbm.at[pl.ds(r0, CHUNK)], xv, sx),
          pltpu.async_copy(d_hbm.at[pl.ds(r0, CHUNK)], dv, sd),
      )

    for cc in range(NBUF):
      issue_in(cc)

    cols = [lane + o for o in GROUP_OFFS]

    for cc in range(NCHUNKS):
      xv, dv, ov, _, _, so = bufs[cc % NBUF]
      cpx, cpd = pend_in.pop(cc)
      cpx.wait()
      cpd.wait()
      if cc - NBUF >= 0:
        pend_out.pop(cc - NBUF).wait()

      @plsc.parallel_loop(0, CHUNK, unroll=2)
      def _(r):
        rsplat = jnp.zeros((L,), jnp.int32) + r
        for col in cols:
          idx = plsc.load_gather(dv, [rsplat, col])
          vals = plsc.load_gather(xv, [rsplat, idx])
          plsc.store_scatter(ov, [rsplat, col], vals)

      r0 = base + cc * CHUNK
      pend_out[cc] = pltpu.async_copy(ov, o_hbm.at[pl.ds(r0, CHUNK)], so)
      if cc + NBUF < NCHUNKS:
        issue_in(cc + NBUF)

    for cc in range(NCHUNKS - NBUF, NCHUNKS):
      pend_out.pop(cc).wait()

  return k(x, d)


def kernel(x, d):
  return _gather_sc(x, d.astype(jnp.int32))
